# Initial kernel scaffold; baseline (speedup 1.0000x reference)
import jax
import jax.numpy as jnp
from jax.experimental import pallas as pl


def kernel(x, W, b, gate_temperature):
    raise NotImplementedError("write your pallas kernel here")


# speedup vs baseline: 1.2701x; 1.2701x over previous
import jax
import jax.numpy as jnp
from jax.experimental import pallas as pl
from jax.experimental.pallas import tpu as pltpu

D_MODEL = 4096
E = 64
K = 8
BLK = 512


def _body(t_ref, x_ref, wt_ref, b_ref, logits_ref, idx_ref, loss_ref, cnt):
    i = pl.program_id(0)
    nsteps = pl.num_programs(0)
    logits = jnp.dot(x_ref[...], wt_ref[...], preferred_element_type=jnp.float32)
    logits = (logits + b_ref[...]) / t_ref[0]
    logits_ref[...] = logits

    iota = jax.lax.broadcasted_iota(jnp.int32, (BLK, E), 1)
    cur = logits
    cols = []
    for j in range(K):
        m = jnp.max(cur, axis=1, keepdims=True)
        idx = jnp.min(jnp.where(cur == m, iota, E), axis=1, keepdims=True)
        cols.append(idx)
        if j == 0:
            @pl.when(i == 0)
            def _():
                cnt[...] = jnp.zeros_like(cnt)

            cnt[0, :] += jnp.sum((iota == idx).astype(jnp.float32), axis=0)
        if j < K - 1:
            cur = jnp.where(iota == idx, -jnp.inf, cur)
    idx_ref[...] = jnp.concatenate(cols, axis=1)

    @pl.when(i == nsteps - 1)
    def _():
        counts = cnt[0, :]
        n_tokens = nsteps * BLK
        actual = counts / n_tokens + 1e-8
        actual = actual / jnp.sum(actual)
        u = 1.0 / E
        kl = jnp.sum(u * (jnp.log(u) - jnp.log(actual)))
        loss_ref[...] = jnp.full((1, 1), 0.1 * kl, dtype=jnp.float32)


def kernel(x, W, b, gate_temperature):
    B, S, D = x.shape
    N = B * S
    xf = x.reshape(N, D)
    wt = W.T
    b2 = b.reshape(1, E)
    grid = N // BLK
    logits, idxs, loss = pl.pallas_call(
        _body,
        grid=(grid,),
        in_specs=[
            pl.BlockSpec(memory_space=pltpu.SMEM),
            pl.BlockSpec((BLK, D), lambda i: (i, 0)),
            pl.BlockSpec((D, E), lambda i: (0, 0)),
            pl.BlockSpec((1, E), lambda i: (0, 0)),
        ],
        out_specs=(
            pl.BlockSpec((BLK, E), lambda i: (i, 0)),
            pl.BlockSpec((BLK, K), lambda i: (i, 0)),
            pl.BlockSpec((1, 1), lambda i: (0, 0)),
        ),
        out_shape=(
            jax.ShapeDtypeStruct((N, E), jnp.float32),
            jax.ShapeDtypeStruct((N, K), jnp.int32),
            jax.ShapeDtypeStruct((1, 1), jnp.float32),
        ),
        scratch_shapes=[pltpu.VMEM((1, E), jnp.float32)],
    )(gate_temperature, xf, wt, b2)
    return logits, idxs, loss.reshape(())
